# baseline (speedup 1.0000x reference)
import functools

import jax
import jax.numpy as jnp
from jax import lax
from jax.experimental import pallas as pl
from jax.experimental.pallas import tpu as pltpu
from jax.experimental.pallas import tpu_sc as plsc

NC = 2
NS = 16
NW = NC * NS
L = 16
EMBED = 32
CHUNK = 128
BATCH = 16384
B_PER_W = BATCH // NW
NCH = B_PER_W // CHUNK


def _sc_body(users_hbm, books_hbm, ut_hbm, bt_hbm, w_hbm, bias_hbm, out_hbm,
             uidx, bidx, urows, brows, wv, biasv, outv, sem):
    wid = lax.axis_index("s") * NC + lax.axis_index("c")

    pltpu.sync_copy(users_hbm.at[wid], uidx)
    pltpu.sync_copy(books_hbm.at[wid], bidx)
    pltpu.sync_copy(w_hbm, wv)
    pltpu.sync_copy(bias_hbm, biasv)

    cps = []
    for j in range(NCH):
        cps.append(pltpu.async_copy(
            ut_hbm.at[uidx.at[j]], urows.at[pl.ds(j * CHUNK, CHUNK)], sem))
        cps.append(pltpu.async_copy(
            bt_hbm.at[bidx.at[j]], brows.at[pl.ds(j * CHUNK, CHUNK)], sem))
    for cp in cps:
        cp.wait()

    bias_vec = biasv[...]

    def group(g, carry):
        rid = g * L + lax.iota(jnp.int32, L)
        acc = bias_vec
        for d in range(EMBED):
            col = jnp.full((L,), d, jnp.int32)
            acc = acc + plsc.load_gather(urows, [rid, col]) * wv[pl.ds(d * L, L)]
            acc = acc + plsc.load_gather(brows, [rid, col]) * wv[pl.ds((EMBED + d) * L, L)]
        outv[pl.ds(g * L, L)] = acc
        return carry

    lax.fori_loop(0, B_PER_W // L, group, 0)
    pltpu.sync_copy(outv, out_hbm.at[pl.ds(wid * B_PER_W, B_PER_W)])


def kernel(users, books, user_table, book_table, W, b):
    batch = users.shape[0]
    users_r = users.astype(jnp.int32).reshape(NW, NCH, CHUNK)
    books_r = books.astype(jnp.int32).reshape(NW, NCH, CHUNK)
    w_bcast = jnp.broadcast_to(W.reshape(2 * EMBED, 1), (2 * EMBED, L)).reshape(-1)
    bias_vec = jnp.broadcast_to(b.reshape(1), (L,))

    mesh = plsc.VectorSubcoreMesh(core_axis_name="c", subcore_axis_name="s")
    fn = functools.partial(
        pl.kernel,
        out_type=jax.ShapeDtypeStruct((batch,), jnp.float32),
        mesh=mesh,
        scratch_types=[
            pltpu.VMEM((NCH, CHUNK), jnp.int32),
            pltpu.VMEM((NCH, CHUNK), jnp.int32),
            pltpu.VMEM((B_PER_W, EMBED), jnp.float32),
            pltpu.VMEM((B_PER_W, EMBED), jnp.float32),
            pltpu.VMEM((2 * EMBED * L,), jnp.float32),
            pltpu.VMEM((L,), jnp.float32),
            pltpu.VMEM((B_PER_W,), jnp.float32),
            pltpu.SemaphoreType.DMA,
        ],
        compiler_params=pltpu.CompilerParams(
            needs_layout_passes=False, use_tc_tiling_on_sc=False),
    )(_sc_body)
    out = fn(users_r, books_r, user_table, book_table, w_bcast, bias_vec)
    return out.reshape(batch, 1)
